# baseline (speedup 1.0000x reference)
import functools

import jax
import jax.numpy as jnp
from jax import lax
from jax.experimental import pallas as pl
from jax.experimental.pallas import tpu as pltpu
from jax.experimental.pallas import tpu_sc as plsc

N_NODES = 2048
N_EDGES = 32768
EMB = 16

NC = 2
NS = 16
NW = NC * NS
NODES_PER_W = N_NODES // NW
EDGES_PER_W = N_EDGES // NW
CHUNK = 128
NCHUNK = EDGES_PER_W // CHUNK
ROWS_PER_SUB = N_NODES // NS
LANES = 16

_mesh = plsc.VectorSubcoreMesh(
    core_axis_name="c", subcore_axis_name="s", num_cores=NC, num_subcores=NS
)


@functools.partial(
    pl.kernel,
    out_type=jax.ShapeDtypeStruct((NC, N_NODES, EMB), jnp.float32),
    mesh=_mesh,
    compiler_params=pltpu.CompilerParams(
        needs_layout_passes=False, use_tc_tiling_on_sc=False
    ),
    scratch_types=[
        pltpu.VMEM((N_NODES,), jnp.int32),
        pltpu.VMEM((NODES_PER_W,), jnp.int32),
        pltpu.VMEM((NODES_PER_W, EMB), jnp.float32),
        pltpu.VMEM((NCHUNK, CHUNK), jnp.int32),
        pltpu.VMEM((NCHUNK, CHUNK), jnp.int32),
        pltpu.VMEM((NCHUNK, CHUNK), jnp.int32),
        pltpu.VMEM((EDGES_PER_W, EMB), jnp.float32),
        pltpu.VMEM((ROWS_PER_SUB, EMB), jnp.float32),
        pltpu.VMEM_SHARED((N_NODES, EMB), jnp.float32),
        pltpu.SemaphoreType.DMA,
        pltpu.SemaphoreType.DMA,
        pltpu.SemaphoreType.DMA,
        pltpu.SemaphoreType.DMA,
    ],
)
def _sc_stage(nf_hbm, src_hbm, dst_hbm, table_hbm, agg_out,
              nf_v, nid_v, x_chunk, src_v, dst_v, gidx_v, rows_v, zero_v,
              agg_sh, sem_in, sem_x, sem_rows, sem_sc):
    c = lax.axis_index("c")
    s = lax.axis_index("s")
    wid = s * NC + c
    base_n = wid * NODES_PER_W
    base_e = wid * NCHUNK

    cp_nf = pltpu.async_copy(nf_hbm, nf_v, sem_in)
    cp_src = pltpu.async_copy(src_hbm.at[pl.ds(base_e, NCHUNK)], src_v, sem_in)
    cp_dst = pltpu.async_copy(dst_hbm.at[pl.ds(base_e, NCHUNK)], dst_v, sem_in)

    def _zero_row(r, carry):
        zero_v[r, :] = jnp.zeros((LANES,), jnp.float32)
        return carry
    lax.fori_loop(0, ROWS_PER_SUB, _zero_row, 0)
    pltpu.sync_copy(zero_v, agg_sh.at[pl.ds(s * ROWS_PER_SUB, ROWS_PER_SUB)])

    for k in range(NODES_PER_W // LANES):
        nid_v[pl.ds(k * LANES, LANES)] = (
            base_n + k * LANES + lax.broadcasted_iota(jnp.int32, (LANES,), 0)
        )

    cp_nf.wait()
    cp_src.wait()
    cp_dst.wait()

    cp_x = pltpu.async_copy(
        table_hbm.at[nf_v.at[pl.ds(base_n, NODES_PER_W)]], x_chunk, sem_x
    )

    for j in range(NCHUNK):
        for i in range(CHUNK // LANES):
            sidx = src_v[j, pl.ds(i * LANES, LANES)]
            gidx_v[j, pl.ds(i * LANES, LANES)] = plsc.load_gather(nf_v, [sidx])

    row_cps = [
        pltpu.async_copy(
            table_hbm.at[gidx_v.at[j]],
            rows_v.at[pl.ds(j * CHUNK, CHUNK)],
            sem_rows,
        )
        for j in range(NCHUNK)
    ]

    plsc.subcore_barrier()

    cp_x.wait()
    sc_x = pltpu.async_copy(x_chunk, agg_sh.at[nid_v], sem_sc, add=True)
    for cp in row_cps:
        cp.wait()
    sc_cps = [
        pltpu.async_copy(
            rows_v.at[pl.ds(j * CHUNK, CHUNK)],
            agg_sh.at[dst_v.at[j]],
            sem_sc,
            add=True,
        )
        for j in range(NCHUNK)
    ]
    sc_x.wait()
    for cp in sc_cps:
        cp.wait()

    plsc.subcore_barrier()

    @pl.when(s == 0)
    def _():
        pltpu.sync_copy(agg_sh, agg_out.at[c])


BLK = 1024
NBLK = N_NODES // BLK


def _tc_body(agg_ref, wei_ref, bei_ref, wat_ref, bat_ref, out_ref, ei_s):
    i = pl.program_id(0)

    @pl.when(i == 0)
    def _():
        h = agg_ref[0] + agg_ref[1]
        ei_s[...] = lax.dot_general(
            h, wei_ref[...], (((1,), (1,)), ((), ())),
            preferred_element_type=jnp.float32,
        ) + bei_ref[...]

    @pl.when(i < NBLK)
    def _():
        ei = ei_s[...]
        ei_blk = ei_s[pl.ds(i * BLK, BLK), :]
        gram = lax.dot_general(
            ei_blk, ei, (((1,), (1,)), ((), ())),
            preferred_element_type=jnp.float32,
        )
        out_ref[...] = gram.reshape(1, BLK * N_NODES)

    @pl.when(i == NBLK)
    def _():
        h0 = agg_ref[0, 0:1, :] + agg_ref[1, 0:1, :]
        at0 = lax.dot_general(
            h0, wat_ref[...], (((1,), (1,)), ((), ())),
            preferred_element_type=jnp.float32,
        ) + bat_ref[...]
        at = jnp.mean(at0, axis=-1, keepdims=True)
        out_ref[...] = jnp.broadcast_to(at, (1, BLK * N_NODES))


def _tc_stage(agg, W_ei, b_ei, W_at, b_at):
    return pl.pallas_call(
        _tc_body,
        grid=(NBLK + 1,),
        in_specs=[
            pl.BlockSpec((NC, N_NODES, EMB), lambda i: (0, 0, 0)),
            pl.BlockSpec((EMB, EMB), lambda i: (0, 0)),
            pl.BlockSpec((1, EMB), lambda i: (0, 0)),
            pl.BlockSpec((EMB, EMB), lambda i: (0, 0)),
            pl.BlockSpec((1, EMB), lambda i: (0, 0)),
        ],
        out_specs=pl.BlockSpec((1, BLK * N_NODES), lambda i: (0, i)),
        out_shape=jax.ShapeDtypeStruct((1, N_NODES * N_NODES + 1),
                                       jnp.float32),
        scratch_shapes=[pltpu.VMEM((N_NODES, EMB), jnp.float32)],
    )(agg, W_ei, b_ei, W_at, b_at)


def kernel(node_feature, batch_ptr, edge_index, node_index, batch_shape,
           emb_table, W_at, b_at, W_ei, b_ei):
    src = edge_index[:, 0].reshape(NW * NCHUNK, CHUNK)
    dst = edge_index[:, 1].reshape(NW * NCHUNK, CHUNK)
    agg = _sc_stage(node_feature, src, dst, emb_table)
    return _tc_stage(
        agg, W_ei, b_ei.reshape(1, EMB), W_at, b_at.reshape(1, EMB)
    )


# speedup vs baseline: 1.0908x; 1.0908x over previous
import functools

import jax
import jax.numpy as jnp
from jax import lax
from jax.experimental import pallas as pl
from jax.experimental.pallas import tpu as pltpu
from jax.experimental.pallas import tpu_sc as plsc

N_NODES = 2048
N_EDGES = 32768
EMB = 16

NC = 2
NS = 16
NW = NC * NS
EDGES_PER_W = N_EDGES // NW
CHUNK = 128
NCHUNK = EDGES_PER_W // CHUNK
ROWS_PER_SUB = N_NODES // NS
LANES = 16

_mesh = plsc.VectorSubcoreMesh(
    core_axis_name="c", subcore_axis_name="s", num_cores=NC, num_subcores=NS
)


@functools.partial(
    pl.kernel,
    out_type=jax.ShapeDtypeStruct((NC, N_NODES, EMB), jnp.float32),
    mesh=_mesh,
    compiler_params=pltpu.CompilerParams(
        needs_layout_passes=False, use_tc_tiling_on_sc=False
    ),
    scratch_types=[
        pltpu.VMEM((ROWS_PER_SUB,), jnp.int32),
        pltpu.VMEM((ROWS_PER_SUB, EMB), jnp.float32),
        pltpu.VMEM((NCHUNK, CHUNK), jnp.int32),
        pltpu.VMEM((NCHUNK, CHUNK), jnp.int32),
        pltpu.VMEM((EDGES_PER_W, EMB), jnp.float32),
        pltpu.VMEM((ROWS_PER_SUB, EMB), jnp.float32),
        pltpu.VMEM_SHARED((N_NODES, EMB), jnp.float32),
        pltpu.VMEM_SHARED((N_NODES, EMB), jnp.float32),
        pltpu.SemaphoreType.DMA,
        pltpu.SemaphoreType.DMA,
        pltpu.SemaphoreType.DMA,
        pltpu.SemaphoreType.DMA,
    ],
)
def _sc_stage(nf_hbm, src_hbm, dst_hbm, table_hbm, agg_out,
              nidx_v, x_rows, src_v, dst_v, rows_v, zero_v, x_sh, agg_sh,
              sem_in, sem_x, sem_rows, sem_sc):
    c = lax.axis_index("c")
    s = lax.axis_index("s")
    wid = s * NC + c
    base_r = s * ROWS_PER_SUB
    base_e = wid * NCHUNK

    cp_nf = pltpu.async_copy(
        nf_hbm.at[pl.ds(base_r, ROWS_PER_SUB)], nidx_v, sem_in
    )
    cp_src = pltpu.async_copy(src_hbm.at[pl.ds(base_e, NCHUNK)], src_v, sem_in)
    cp_dst = pltpu.async_copy(dst_hbm.at[pl.ds(base_e, NCHUNK)], dst_v, sem_in)

    def _zero_row(r, carry):
        zero_v[r, :] = jnp.zeros((LANES,), jnp.float32)
        return carry
    lax.fori_loop(0, ROWS_PER_SUB, _zero_row, 0)

    cp_nf.wait()
    pltpu.async_copy(table_hbm.at[nidx_v], x_rows, sem_x).wait()

    cp_xs = pltpu.async_copy(x_rows, x_sh.at[pl.ds(base_r, ROWS_PER_SUB)],
                             sem_x)
    @pl.when(c == 0)
    def _():
        pltpu.sync_copy(x_rows, agg_sh.at[pl.ds(base_r, ROWS_PER_SUB)])

    @pl.when(c != 0)
    def _():
        pltpu.sync_copy(zero_v, agg_sh.at[pl.ds(base_r, ROWS_PER_SUB)])
    cp_xs.wait()
    cp_src.wait()
    cp_dst.wait()

    plsc.subcore_barrier()

    row_cps = [
        pltpu.async_copy(
            x_sh.at[src_v.at[j]],
            rows_v.at[pl.ds(j * CHUNK, CHUNK)],
            sem_rows,
        )
        for j in range(NCHUNK)
    ]
    for cp in row_cps:
        cp.wait()
    sc_cps = [
        pltpu.async_copy(
            rows_v.at[pl.ds(j * CHUNK, CHUNK)],
            agg_sh.at[dst_v.at[j]],
            sem_sc,
            add=True,
        )
        for j in range(NCHUNK)
    ]
    for cp in sc_cps:
        cp.wait()

    plsc.subcore_barrier()

    @pl.when(s == 0)
    def _():
        pltpu.sync_copy(agg_sh, agg_out.at[c])


BLK = 512
NBLK = N_NODES // BLK


def _tc_body(agg_ref, wei_ref, bei_ref, wat_ref, bat_ref, out_ref, ei_s):
    i = pl.program_id(0)

    @pl.when(i == 0)
    def _():
        h = agg_ref[0] + agg_ref[1]
        ei_s[...] = lax.dot_general(
            h, wei_ref[...], (((1,), (1,)), ((), ())),
            preferred_element_type=jnp.float32,
        ) + bei_ref[...]

    @pl.when(i < NBLK)
    def _():
        ei = ei_s[...]
        ei_blk = ei_s[pl.ds(i * BLK, BLK), :]
        gram = lax.dot_general(
            ei_blk, ei, (((1,), (1,)), ((), ())),
            preferred_element_type=jnp.float32,
        )
        out_ref[...] = gram.reshape(1, BLK * N_NODES)

    @pl.when(i == NBLK)
    def _():
        h0 = agg_ref[0, 0:1, :] + agg_ref[1, 0:1, :]
        at0 = lax.dot_general(
            h0, wat_ref[...], (((1,), (1,)), ((), ())),
            preferred_element_type=jnp.float32,
        ) + bat_ref[...]
        at = jnp.mean(at0, axis=-1, keepdims=True)
        out_ref[...] = jnp.broadcast_to(at, (1, BLK * N_NODES))


def _tc_stage(agg, W_ei, b_ei, W_at, b_at):
    return pl.pallas_call(
        _tc_body,
        grid=(NBLK + 1,),
        in_specs=[
            pl.BlockSpec((NC, N_NODES, EMB), lambda i: (0, 0, 0)),
            pl.BlockSpec((EMB, EMB), lambda i: (0, 0)),
            pl.BlockSpec((1, EMB), lambda i: (0, 0)),
            pl.BlockSpec((EMB, EMB), lambda i: (0, 0)),
            pl.BlockSpec((1, EMB), lambda i: (0, 0)),
        ],
        out_specs=pl.BlockSpec((1, BLK * N_NODES), lambda i: (0, i)),
        out_shape=jax.ShapeDtypeStruct((1, N_NODES * N_NODES + 1),
                                       jnp.float32),
        scratch_shapes=[pltpu.VMEM((N_NODES, EMB), jnp.float32)],
    )(agg, W_ei, b_ei, W_at, b_at)


def kernel(node_feature, batch_ptr, edge_index, node_index, batch_shape,
           emb_table, W_at, b_at, W_ei, b_ei):
    src = edge_index[:, 0].reshape(NW * NCHUNK, CHUNK)
    dst = edge_index[:, 1].reshape(NW * NCHUNK, CHUNK)
    agg = _sc_stage(node_feature, src, dst, emb_table)
    return _tc_stage(
        agg, W_ei, b_ei.reshape(1, EMB), W_at, b_at.reshape(1, EMB)
    )


# speedup vs baseline: 1.0929x; 1.0019x over previous
import functools

import jax
import jax.numpy as jnp
from jax import lax
from jax.experimental import pallas as pl
from jax.experimental.pallas import tpu as pltpu
from jax.experimental.pallas import tpu_sc as plsc

N_NODES = 2048
N_EDGES = 32768
EMB = 16

NC = 2
NS = 16
NW = NC * NS
EDGES_PER_W = N_EDGES // NW
CHUNK = 128
NCHUNK = EDGES_PER_W // CHUNK
ROWS_PER_SUB = N_NODES // NS
LANES = 16

_mesh = plsc.VectorSubcoreMesh(
    core_axis_name="c", subcore_axis_name="s", num_cores=NC, num_subcores=NS
)


@functools.partial(
    pl.kernel,
    out_type=jax.ShapeDtypeStruct((NC, N_NODES, EMB), jnp.float32),
    mesh=_mesh,
    compiler_params=pltpu.CompilerParams(
        needs_layout_passes=False, use_tc_tiling_on_sc=False
    ),
    scratch_types=[
        pltpu.VMEM((ROWS_PER_SUB,), jnp.int32),
        pltpu.VMEM((ROWS_PER_SUB, EMB), jnp.float32),
        pltpu.VMEM((NCHUNK, CHUNK), jnp.int32),
        pltpu.VMEM((NCHUNK, CHUNK), jnp.int32),
        pltpu.VMEM((EDGES_PER_W, EMB), jnp.float32),
        pltpu.VMEM((ROWS_PER_SUB, EMB), jnp.float32),
        pltpu.VMEM_SHARED((N_NODES, EMB), jnp.float32),
        pltpu.VMEM_SHARED((N_NODES, EMB), jnp.float32),
        pltpu.SemaphoreType.DMA,
        pltpu.SemaphoreType.DMA,
        pltpu.SemaphoreType.DMA,
    ] + [pltpu.SemaphoreType.DMA] * NCHUNK,
)
def _sc_stage(nf_hbm, src_hbm, dst_hbm, table_hbm, agg_out,
              nidx_v, x_rows, src_v, dst_v, rows_v, zero_v, x_sh, agg_sh,
              sem_in, sem_x, sem_sc, *sem_rows):
    c = lax.axis_index("c")
    s = lax.axis_index("s")
    wid = s * NC + c
    base_r = s * ROWS_PER_SUB
    base_e = wid * NCHUNK

    cp_nf = pltpu.async_copy(
        nf_hbm.at[pl.ds(base_r, ROWS_PER_SUB)], nidx_v, sem_in
    )
    cp_src = pltpu.async_copy(src_hbm.at[pl.ds(base_e, NCHUNK)], src_v, sem_in)
    cp_dst = pltpu.async_copy(dst_hbm.at[pl.ds(base_e, NCHUNK)], dst_v, sem_in)

    for r in range(ROWS_PER_SUB):
        zero_v[r, :] = jnp.zeros((LANES,), jnp.float32)

    cp_nf.wait()
    pltpu.async_copy(table_hbm.at[nidx_v], x_rows, sem_x).wait()

    cp_xs = pltpu.async_copy(x_rows, x_sh.at[pl.ds(base_r, ROWS_PER_SUB)],
                             sem_x)
    @pl.when(c == 0)
    def _():
        pltpu.sync_copy(x_rows, agg_sh.at[pl.ds(base_r, ROWS_PER_SUB)])

    @pl.when(c != 0)
    def _():
        pltpu.sync_copy(zero_v, agg_sh.at[pl.ds(base_r, ROWS_PER_SUB)])
    cp_xs.wait()
    cp_src.wait()
    cp_dst.wait()

    plsc.subcore_barrier()

    row_cps = [
        pltpu.async_copy(
            x_sh.at[src_v.at[j]],
            rows_v.at[pl.ds(j * CHUNK, CHUNK)],
            sem_rows[j],
        )
        for j in range(NCHUNK)
    ]
    sc_cps = []
    for j in range(NCHUNK):
        row_cps[j].wait()
        sc_cps.append(pltpu.async_copy(
            rows_v.at[pl.ds(j * CHUNK, CHUNK)],
            agg_sh.at[dst_v.at[j]],
            sem_sc,
            add=True,
        ))
    for cp in sc_cps:
        cp.wait()

    plsc.subcore_barrier()

    pltpu.sync_copy(
        agg_sh.at[pl.ds(base_r, ROWS_PER_SUB)],
        agg_out.at[c, pl.ds(base_r, ROWS_PER_SUB)],
    )


BLK = 512
NBLK = N_NODES // BLK


def _tc_body(agg_ref, wei_ref, bei_ref, wat_ref, bat_ref, out_ref, ei_s):
    i = pl.program_id(0)

    @pl.when(i == 0)
    def _():
        h = agg_ref[0] + agg_ref[1]
        ei_s[...] = lax.dot_general(
            h, wei_ref[...], (((1,), (1,)), ((), ())),
            preferred_element_type=jnp.float32,
        ) + bei_ref[...]

    @pl.when(i < NBLK)
    def _():
        ei = ei_s[...]
        ei_blk = ei_s[pl.ds(i * BLK, BLK), :]
        gram = lax.dot_general(
            ei_blk, ei, (((1,), (1,)), ((), ())),
            preferred_element_type=jnp.float32,
        )
        out_ref[...] = gram.reshape(1, BLK * N_NODES)

    @pl.when(i == NBLK)
    def _():
        h0 = agg_ref[0, 0:1, :] + agg_ref[1, 0:1, :]
        at0 = lax.dot_general(
            h0, wat_ref[...], (((1,), (1,)), ((), ())),
            preferred_element_type=jnp.float32,
        ) + bat_ref[...]
        at = jnp.mean(at0, axis=-1, keepdims=True)
        out_ref[...] = jnp.broadcast_to(at, (1, BLK * N_NODES))


def _tc_stage(agg, W_ei, b_ei, W_at, b_at):
    return pl.pallas_call(
        _tc_body,
        grid=(NBLK + 1,),
        in_specs=[
            pl.BlockSpec((NC, N_NODES, EMB), lambda i: (0, 0, 0)),
            pl.BlockSpec((EMB, EMB), lambda i: (0, 0)),
            pl.BlockSpec((1, EMB), lambda i: (0, 0)),
            pl.BlockSpec((EMB, EMB), lambda i: (0, 0)),
            pl.BlockSpec((1, EMB), lambda i: (0, 0)),
        ],
        out_specs=pl.BlockSpec((1, BLK * N_NODES), lambda i: (0, i)),
        out_shape=jax.ShapeDtypeStruct((1, N_NODES * N_NODES + 1),
                                       jnp.float32),
        scratch_shapes=[pltpu.VMEM((N_NODES, EMB), jnp.float32)],
    )(agg, W_ei, b_ei, W_at, b_at)


def kernel(node_feature, batch_ptr, edge_index, node_index, batch_shape,
           emb_table, W_at, b_at, W_ei, b_ei):
    src = edge_index[:, 0].reshape(NW * NCHUNK, CHUNK)
    dst = edge_index[:, 1].reshape(NW * NCHUNK, CHUNK)
    agg = _sc_stage(node_feature, src, dst, emb_table)
    return _tc_stage(
        agg, W_ei, b_ei.reshape(1, EMB), W_at, b_at.reshape(1, EMB)
    )


# speedup vs baseline: 1.1274x; 1.0316x over previous
import functools

import jax
import jax.numpy as jnp
from jax import lax
from jax.experimental import pallas as pl
from jax.experimental.pallas import tpu as pltpu
from jax.experimental.pallas import tpu_sc as plsc

N_NODES = 2048
N_EDGES = 32768
EMB = 16

NC = 1
NS = 16
NW = NC * NS
EDGES_PER_W = N_EDGES // NW
CHUNK = 128
NCHUNK = EDGES_PER_W // CHUNK
ROWS_PER_SUB = N_NODES // NS
LANES = 16

_mesh = plsc.VectorSubcoreMesh(
    core_axis_name="c", subcore_axis_name="s", num_cores=NC, num_subcores=NS
)


@functools.partial(
    pl.kernel,
    out_type=jax.ShapeDtypeStruct((NC, N_NODES, EMB), jnp.float32),
    mesh=_mesh,
    compiler_params=pltpu.CompilerParams(
        needs_layout_passes=False, use_tc_tiling_on_sc=False
    ),
    scratch_types=[
        pltpu.VMEM((ROWS_PER_SUB,), jnp.int32),
        pltpu.VMEM((ROWS_PER_SUB, EMB), jnp.float32),
        pltpu.VMEM((NCHUNK, CHUNK), jnp.int32),
        pltpu.VMEM((NCHUNK, CHUNK), jnp.int32),
        pltpu.VMEM((EDGES_PER_W, EMB), jnp.float32),
        pltpu.VMEM((ROWS_PER_SUB, EMB), jnp.float32),
        pltpu.VMEM_SHARED((N_NODES, EMB), jnp.float32),
        pltpu.VMEM_SHARED((N_NODES, EMB), jnp.float32),
        pltpu.SemaphoreType.DMA,
        pltpu.SemaphoreType.DMA,
        pltpu.SemaphoreType.DMA,
    ] + [pltpu.SemaphoreType.DMA] * NCHUNK,
)
def _sc_stage(nf_hbm, src_hbm, dst_hbm, table_hbm, agg_out,
              nidx_v, x_rows, src_v, dst_v, rows_v, zero_v, x_sh, agg_sh,
              sem_in, sem_x, sem_sc, *sem_rows):
    c = lax.axis_index("c")
    s = lax.axis_index("s")
    wid = s * NC + c
    base_r = s * ROWS_PER_SUB
    base_e = wid * NCHUNK

    cp_nf = pltpu.async_copy(
        nf_hbm.at[pl.ds(base_r, ROWS_PER_SUB)], nidx_v, sem_in
    )
    cp_src = pltpu.async_copy(src_hbm.at[pl.ds(base_e, NCHUNK)], src_v, sem_in)
    cp_dst = pltpu.async_copy(dst_hbm.at[pl.ds(base_e, NCHUNK)], dst_v, sem_in)

    for r in range(ROWS_PER_SUB):
        zero_v[r, :] = jnp.zeros((LANES,), jnp.float32)

    cp_nf.wait()
    pltpu.async_copy(table_hbm.at[nidx_v], x_rows, sem_x).wait()

    cp_xs = pltpu.async_copy(x_rows, x_sh.at[pl.ds(base_r, ROWS_PER_SUB)],
                             sem_x)
    @pl.when(c == 0)
    def _():
        pltpu.sync_copy(x_rows, agg_sh.at[pl.ds(base_r, ROWS_PER_SUB)])

    @pl.when(c != 0)
    def _():
        pltpu.sync_copy(zero_v, agg_sh.at[pl.ds(base_r, ROWS_PER_SUB)])
    cp_xs.wait()
    cp_src.wait()
    cp_dst.wait()

    plsc.subcore_barrier()

    row_cps = [
        pltpu.async_copy(
            x_sh.at[src_v.at[j]],
            rows_v.at[pl.ds(j * CHUNK, CHUNK)],
            sem_rows[j],
        )
        for j in range(NCHUNK)
    ]
    sc_cps = []
    for j in range(NCHUNK):
        row_cps[j].wait()
        sc_cps.append(pltpu.async_copy(
            rows_v.at[pl.ds(j * CHUNK, CHUNK)],
            agg_sh.at[dst_v.at[j]],
            sem_sc,
            add=True,
        ))
    for cp in sc_cps:
        cp.wait()

    plsc.subcore_barrier()

    pltpu.sync_copy(
        agg_sh.at[pl.ds(base_r, ROWS_PER_SUB)],
        agg_out.at[c, pl.ds(base_r, ROWS_PER_SUB)],
    )


BLK = 512
NBLK = N_NODES // BLK


def _tc_body(agg_ref, wei_ref, bei_ref, wat_ref, bat_ref, out_ref, ei_s):
    i = pl.program_id(0)

    @pl.when(i == 0)
    def _():
        h = agg_ref[0]
        ei_s[...] = lax.dot_general(
            h, wei_ref[...], (((1,), (1,)), ((), ())),
            preferred_element_type=jnp.float32,
        ) + bei_ref[...]

    @pl.when(i < NBLK)
    def _():
        ei = ei_s[...]
        ei_blk = ei_s[pl.ds(i * BLK, BLK), :]
        gram = lax.dot_general(
            ei_blk, ei, (((1,), (1,)), ((), ())),
            preferred_element_type=jnp.float32,
        )
        out_ref[...] = gram.reshape(1, BLK * N_NODES)

    @pl.when(i == NBLK)
    def _():
        h0 = agg_ref[0, 0:1, :]
        at0 = lax.dot_general(
            h0, wat_ref[...], (((1,), (1,)), ((), ())),
            preferred_element_type=jnp.float32,
        ) + bat_ref[...]
        at = jnp.mean(at0, axis=-1, keepdims=True)
        out_ref[...] = jnp.broadcast_to(at, (1, BLK * N_NODES))


def _tc_stage(agg, W_ei, b_ei, W_at, b_at):
    return pl.pallas_call(
        _tc_body,
        grid=(NBLK + 1,),
        in_specs=[
            pl.BlockSpec((NC, N_NODES, EMB), lambda i: (0, 0, 0)),
            pl.BlockSpec((EMB, EMB), lambda i: (0, 0)),
            pl.BlockSpec((1, EMB), lambda i: (0, 0)),
            pl.BlockSpec((EMB, EMB), lambda i: (0, 0)),
            pl.BlockSpec((1, EMB), lambda i: (0, 0)),
        ],
        out_specs=pl.BlockSpec((1, BLK * N_NODES), lambda i: (0, i)),
        out_shape=jax.ShapeDtypeStruct((1, N_NODES * N_NODES + 1),
                                       jnp.float32),
        scratch_shapes=[pltpu.VMEM((N_NODES, EMB), jnp.float32)],
    )(agg, W_ei, b_ei, W_at, b_at)


def kernel(node_feature, batch_ptr, edge_index, node_index, batch_shape,
           emb_table, W_at, b_at, W_ei, b_ei):
    src = edge_index[:, 0].reshape(NW * NCHUNK, CHUNK)
    dst = edge_index[:, 1].reshape(NW * NCHUNK, CHUNK)
    agg = _sc_stage(node_feature, src, dst, emb_table)
    return _tc_stage(
        agg, W_ei, b_ei.reshape(1, EMB), W_at, b_at.reshape(1, EMB)
    )


# speedup vs baseline: 1.1918x; 1.0571x over previous
import functools

import jax
import jax.numpy as jnp
from jax import lax
from jax.experimental import pallas as pl
from jax.experimental.pallas import tpu as pltpu
from jax.experimental.pallas import tpu_sc as plsc

N_NODES = 2048
N_EDGES = 32768
EMB = 16

NC = 1
NS = 16
NW = NC * NS
EDGES_PER_W = N_EDGES // NW
CHUNK = 128
NCHUNK = EDGES_PER_W // CHUNK
ROWS_PER_SUB = N_NODES // NS
LANES = 16

_mesh = plsc.VectorSubcoreMesh(
    core_axis_name="c", subcore_axis_name="s", num_cores=NC, num_subcores=NS
)


@functools.partial(
    pl.kernel,
    out_type=jax.ShapeDtypeStruct((N_NODES, 128), jnp.float32),
    mesh=_mesh,
    compiler_params=pltpu.CompilerParams(
        needs_layout_passes=False, use_tc_tiling_on_sc=False
    ),
    scratch_types=[
        pltpu.VMEM((ROWS_PER_SUB,), jnp.int32),
        pltpu.VMEM((ROWS_PER_SUB, EMB), jnp.float32),
        pltpu.VMEM((NCHUNK, CHUNK), jnp.int32),
        pltpu.VMEM((NCHUNK, CHUNK), jnp.int32),
        pltpu.VMEM((EDGES_PER_W, EMB), jnp.float32),
        pltpu.VMEM((ROWS_PER_SUB, EMB), jnp.float32),
        pltpu.VMEM_SHARED((N_NODES, EMB), jnp.float32),
        pltpu.VMEM_SHARED((N_NODES, EMB), jnp.float32),
        pltpu.SemaphoreType.DMA,
        pltpu.SemaphoreType.DMA,
        pltpu.SemaphoreType.DMA,
    ] + [pltpu.SemaphoreType.DMA] * NCHUNK,
)
def _sc_stage(nf_hbm, src_hbm, dst_hbm, table_hbm, agg_out,
              nidx_v, x_rows, src_v, dst_v, rows_v, zero_v, x_sh, agg_sh,
              sem_in, sem_x, sem_sc, *sem_rows):
    c = lax.axis_index("c")
    s = lax.axis_index("s")
    wid = s * NC + c
    base_r = s * ROWS_PER_SUB
    base_e = wid * NCHUNK

    cp_nf = pltpu.async_copy(
        nf_hbm.at[pl.ds(base_r, ROWS_PER_SUB)], nidx_v, sem_in
    )
    cp_src = pltpu.async_copy(src_hbm.at[pl.ds(base_e, NCHUNK)], src_v, sem_in)
    cp_dst = pltpu.async_copy(dst_hbm.at[pl.ds(base_e, NCHUNK)], dst_v, sem_in)

    for r in range(ROWS_PER_SUB):
        zero_v[r, :] = jnp.zeros((LANES,), jnp.float32)

    cp_nf.wait()
    pltpu.async_copy(table_hbm.at[nidx_v], x_rows, sem_x).wait()

    cp_xs = pltpu.async_copy(x_rows, x_sh.at[pl.ds(base_r, ROWS_PER_SUB)],
                             sem_x)
    @pl.when(c == 0)
    def _():
        pltpu.sync_copy(x_rows, agg_sh.at[pl.ds(base_r, ROWS_PER_SUB)])

    @pl.when(c != 0)
    def _():
        pltpu.sync_copy(zero_v, agg_sh.at[pl.ds(base_r, ROWS_PER_SUB)])
    cp_xs.wait()
    cp_src.wait()
    cp_dst.wait()

    plsc.subcore_barrier()

    row_cps = [
        pltpu.async_copy(
            x_sh.at[src_v.at[j]],
            rows_v.at[pl.ds(j * CHUNK, CHUNK)],
            sem_rows[j],
        )
        for j in range(NCHUNK)
    ]
    sc_cps = []
    for j in range(NCHUNK):
        row_cps[j].wait()
        sc_cps.append(pltpu.async_copy(
            rows_v.at[pl.ds(j * CHUNK, CHUNK)],
            agg_sh.at[dst_v.at[j]],
            sem_sc,
            add=True,
        ))
    for cp in sc_cps:
        cp.wait()

    plsc.subcore_barrier()

    pltpu.sync_copy(
        agg_sh.at[pl.ds(base_r, ROWS_PER_SUB)],
        agg_out.at[pl.ds(base_r, ROWS_PER_SUB), pl.ds(0, EMB)],
    )


BLK = 512
NBLK = N_NODES // BLK


def _tc_body(agg_ref, wei_ref, bei_ref, wat_ref, bat_ref, out_ref, ei_s):
    i = pl.program_id(0)

    @pl.when(i == 0)
    def _():
        h = agg_ref[:, 0:EMB]
        ei_s[...] = lax.dot_general(
            h, wei_ref[...], (((1,), (1,)), ((), ())),
            preferred_element_type=jnp.float32,
        ) + bei_ref[...]

    @pl.when(i < NBLK)
    def _():
        ei = ei_s[...]
        ei_blk = ei_s[pl.ds(i * BLK, BLK), :]
        gram = lax.dot_general(
            ei_blk, ei, (((1,), (1,)), ((), ())),
            preferred_element_type=jnp.float32,
        )
        out_ref[...] = gram.reshape(1, BLK * N_NODES)

    @pl.when(i == NBLK)
    def _():
        h0 = agg_ref[0:1, 0:EMB]
        at0 = lax.dot_general(
            h0, wat_ref[...], (((1,), (1,)), ((), ())),
            preferred_element_type=jnp.float32,
        ) + bat_ref[...]
        at = jnp.mean(at0, axis=-1, keepdims=True)
        out_ref[...] = jnp.broadcast_to(at, (1, BLK * N_NODES))


def _tc_stage(agg, W_ei, b_ei, W_at, b_at):
    return pl.pallas_call(
        _tc_body,
        grid=(NBLK + 1,),
        in_specs=[
            pl.BlockSpec((N_NODES, 128), lambda i: (0, 0)),
            pl.BlockSpec((EMB, EMB), lambda i: (0, 0)),
            pl.BlockSpec((1, EMB), lambda i: (0, 0)),
            pl.BlockSpec((EMB, EMB), lambda i: (0, 0)),
            pl.BlockSpec((1, EMB), lambda i: (0, 0)),
        ],
        out_specs=pl.BlockSpec((1, BLK * N_NODES), lambda i: (0, i)),
        out_shape=jax.ShapeDtypeStruct((1, N_NODES * N_NODES + 1),
                                       jnp.float32),
        scratch_shapes=[pltpu.VMEM((N_NODES, EMB), jnp.float32)],
    )(agg, W_ei, b_ei, W_at, b_at)


def kernel(node_feature, batch_ptr, edge_index, node_index, batch_shape,
           emb_table, W_at, b_at, W_ei, b_ei):
    src = edge_index[:, 0].reshape(NW * NCHUNK, CHUNK)
    dst = edge_index[:, 1].reshape(NW * NCHUNK, CHUNK)
    agg = _sc_stage(node_feature, src, dst, emb_table)
    return _tc_stage(
        agg, W_ei, b_ei.reshape(1, EMB), W_at, b_at.reshape(1, EMB)
    )


# speedup vs baseline: 1.1999x; 1.0068x over previous
import functools

import jax
import jax.numpy as jnp
from jax import lax
from jax.experimental import pallas as pl
from jax.experimental.pallas import tpu as pltpu
from jax.experimental.pallas import tpu_sc as plsc

N_NODES = 2048
N_EDGES = 32768
EMB = 16

NC = 1
NS = 16
NW = NC * NS
EDGES_PER_W = N_EDGES // NW
CHUNK = 128
NCHUNK = EDGES_PER_W // CHUNK
ROWS_PER_SUB = N_NODES // NS
LANES = 16

_mesh = plsc.VectorSubcoreMesh(
    core_axis_name="c", subcore_axis_name="s", num_cores=NC, num_subcores=NS
)


@functools.partial(
    pl.kernel,
    out_type=jax.ShapeDtypeStruct((N_NODES, 128), jnp.float32),
    mesh=_mesh,
    compiler_params=pltpu.CompilerParams(
        needs_layout_passes=False, use_tc_tiling_on_sc=False
    ),
    scratch_types=[
        pltpu.VMEM((ROWS_PER_SUB,), jnp.int32),
        pltpu.VMEM((ROWS_PER_SUB, EMB), jnp.float32),
        pltpu.VMEM((NCHUNK, CHUNK), jnp.int32),
        pltpu.VMEM((NCHUNK, CHUNK), jnp.int32),
        pltpu.VMEM((EDGES_PER_W, EMB), jnp.float32),
        pltpu.VMEM_SHARED((N_NODES, EMB), jnp.float32),
        pltpu.VMEM_SHARED((N_NODES, EMB), jnp.float32),
        pltpu.SemaphoreType.DMA,
        pltpu.SemaphoreType.DMA,
        pltpu.SemaphoreType.DMA,
    ] + [pltpu.SemaphoreType.DMA] * NCHUNK,
)
def _sc_stage(nf_hbm, src_hbm, dst_hbm, table_hbm, agg_out,
              nidx_v, x_rows, src_v, dst_v, rows_v, x_sh, agg_sh,
              sem_in, sem_x, sem_sc, *sem_rows):
    c = lax.axis_index("c")
    s = lax.axis_index("s")
    wid = s * NC + c
    base_r = s * ROWS_PER_SUB
    base_e = wid * NCHUNK

    cp_nf = pltpu.async_copy(
        nf_hbm.at[pl.ds(base_r, ROWS_PER_SUB)], nidx_v, sem_in
    )
    cp_src = pltpu.async_copy(src_hbm.at[pl.ds(base_e, NCHUNK)], src_v, sem_in)
    cp_dst = pltpu.async_copy(dst_hbm.at[pl.ds(base_e, NCHUNK)], dst_v, sem_in)

    cp_nf.wait()
    pltpu.async_copy(table_hbm.at[nidx_v], x_rows, sem_x).wait()

    cp_xs = pltpu.async_copy(x_rows, x_sh.at[pl.ds(base_r, ROWS_PER_SUB)],
                             sem_x)
    pltpu.sync_copy(x_rows, agg_sh.at[pl.ds(base_r, ROWS_PER_SUB)])
    cp_xs.wait()
    cp_src.wait()
    cp_dst.wait()

    plsc.subcore_barrier()

    row_cps = [
        pltpu.async_copy(
            x_sh.at[src_v.at[j]],
            rows_v.at[pl.ds(j * CHUNK, CHUNK)],
            sem_rows[j],
        )
        for j in range(NCHUNK)
    ]
    sc_cps = []
    for j in range(NCHUNK):
        row_cps[j].wait()
        sc_cps.append(pltpu.async_copy(
            rows_v.at[pl.ds(j * CHUNK, CHUNK)],
            agg_sh.at[dst_v.at[j]],
            sem_sc,
            add=True,
        ))
    for cp in sc_cps:
        cp.wait()

    plsc.subcore_barrier()

    pltpu.sync_copy(
        agg_sh.at[pl.ds(base_r, ROWS_PER_SUB)],
        agg_out.at[pl.ds(base_r, ROWS_PER_SUB), pl.ds(0, EMB)],
    )


BLK = 512
NBLK = N_NODES // BLK


def _tc_body(agg_ref, wei_ref, bei_ref, wat_ref, bat_ref, out_ref, ei_s):
    i = pl.program_id(0)

    @pl.when(i == 0)
    def _():
        h = agg_ref[:, 0:EMB]
        ei_s[...] = lax.dot_general(
            h, wei_ref[...], (((1,), (1,)), ((), ())),
            preferred_element_type=jnp.float32,
        ) + bei_ref[...]

    @pl.when(i < NBLK)
    def _():
        ei = ei_s[...]
        ei_blk = ei_s[pl.ds(i * BLK, BLK), :]
        gram = lax.dot_general(
            ei_blk, ei, (((1,), (1,)), ((), ())),
            preferred_element_type=jnp.float32,
        )
        out_ref[...] = gram.reshape(1, BLK * N_NODES)

    @pl.when(i == NBLK)
    def _():
        h0 = agg_ref[0:1, 0:EMB]
        at0 = lax.dot_general(
            h0, wat_ref[...], (((1,), (1,)), ((), ())),
            preferred_element_type=jnp.float32,
        ) + bat_ref[...]
        at = jnp.mean(at0, axis=-1, keepdims=True)
        out_ref[...] = jnp.broadcast_to(at, (1, BLK * N_NODES))


def _tc_stage(agg, W_ei, b_ei, W_at, b_at):
    return pl.pallas_call(
        _tc_body,
        grid=(NBLK + 1,),
        in_specs=[
            pl.BlockSpec((N_NODES, 128), lambda i: (0, 0)),
            pl.BlockSpec((EMB, EMB), lambda i: (0, 0)),
            pl.BlockSpec((1, EMB), lambda i: (0, 0)),
            pl.BlockSpec((EMB, EMB), lambda i: (0, 0)),
            pl.BlockSpec((1, EMB), lambda i: (0, 0)),
        ],
        out_specs=pl.BlockSpec((1, BLK * N_NODES), lambda i: (0, i)),
        out_shape=jax.ShapeDtypeStruct((1, N_NODES * N_NODES + 1),
                                       jnp.float32),
        scratch_shapes=[pltpu.VMEM((N_NODES, EMB), jnp.float32)],
    )(agg, W_ei, b_ei, W_at, b_at)


def kernel(node_feature, batch_ptr, edge_index, node_index, batch_shape,
           emb_table, W_at, b_at, W_ei, b_ei):
    src = edge_index[:, 0].reshape(NW * NCHUNK, CHUNK)
    dst = edge_index[:, 1].reshape(NW * NCHUNK, CHUNK)
    agg = _sc_stage(node_feature, src, dst, emb_table)
    return _tc_stage(
        agg, W_ei, b_ei.reshape(1, EMB), W_at, b_at.reshape(1, EMB)
    )
